# Initial kernel scaffold; baseline (speedup 1.0000x reference)
import jax
import jax.numpy as jnp
from jax.experimental import pallas as pl


def kernel(X, row_missing_idx):
    raise NotImplementedError("write your pallas kernel here")


# speedup vs baseline: 6.9986x; 6.9986x over previous
import functools

import jax
import jax.numpy as jnp
from jax import lax
from jax.experimental import pallas as pl
from jax.experimental.pallas import tpu as pltpu
from jax.experimental.pallas import tpu_sc as plsc

N = 1_000_000
M = 500_000
NC = 2
NS = 16
NW = NC * NS
ROWS_PER_TILE = N // NW
LOCAL_PAD = ((ROWS_PER_TILE + 15) // 16) * 16
CHUNK = 20_000
NCHUNK = M // CHUNK
VPC = CHUNK // 16
L = 16

_mesh = plsc.VectorSubcoreMesh(core_axis_name="c", subcore_axis_name="s")


@functools.partial(
    pl.kernel,
    mesh=_mesh,
    out_type=jax.ShapeDtypeStruct((NW, ROWS_PER_TILE), jnp.int32),
    scratch_types=[
        pltpu.VMEM((CHUNK,), jnp.int32),
        pltpu.VMEM((LOCAL_PAD,), jnp.int32),
    ],
    compiler_params=pltpu.CompilerParams(
        needs_layout_passes=False, use_tc_tiling_on_sc=False),
)
def _scatter_arange(idx_hbm, out_hbm, idx_buf, local):
    wid = lax.axis_index("c") * NS + lax.axis_index("s")
    base = (wid * ROWS_PER_TILE).astype(jnp.int32)
    iota = lax.iota(jnp.int32, L)
    zeros = jnp.zeros((L,), jnp.int32)

    def _zero_body(j, _):
        local[pl.ds(j * L, L)] = zeros
        return 0

    lax.fori_loop(0, LOCAL_PAD // L, _zero_body, 0)

    def _chunk_body(c, _):
        pltpu.sync_copy(idx_hbm.at[pl.ds(c * CHUNK, CHUNK)], idx_buf)
        chunk_base = (c * CHUNK).astype(jnp.int32)

        def _vec_body(v, _):
            ivec = idx_buf[pl.ds(v * L, L)]
            val = (chunk_base + v * L) + iota
            loc = ivec - base
            m = (ivec >= base) & (loc < ROWS_PER_TILE)
            plsc.store_scatter(local, [loc], val, mask=m)
            return 0

        lax.fori_loop(0, VPC, _vec_body, 0)
        return 0

    lax.fori_loop(0, NCHUNK, _chunk_body, 0)
    pltpu.sync_copy(local.at[pl.ds(0, ROWS_PER_TILE)], out_hbm.at[wid])


def kernel(X, row_missing_idx):
    del X
    return _scatter_arange(row_missing_idx).reshape(-1)


# speedup vs baseline: 22.1758x; 3.1686x over previous
import functools

import jax
import jax.numpy as jnp
from jax import lax
from jax.experimental import pallas as pl
from jax.experimental.pallas import tpu as pltpu
from jax.experimental.pallas import tpu_sc as plsc

N = 1_000_000
M = 500_000
NC = 2
NS = 16
NW = NC * NS
ROWS_PER_TILE = N // NW
LOCAL_PAD = ((ROWS_PER_TILE + 15) // 16) * 16
CHUNK = 20_000
NCHUNK = M // CHUNK
VPC = CHUNK // 16
L = 16

_mesh = plsc.VectorSubcoreMesh(core_axis_name="c", subcore_axis_name="s")


@functools.partial(
    pl.kernel,
    mesh=_mesh,
    out_type=jax.ShapeDtypeStruct((NW, ROWS_PER_TILE), jnp.int32),
    scratch_types=[
        pltpu.VMEM((CHUNK,), jnp.int32),
        pltpu.VMEM((LOCAL_PAD,), jnp.int32),
    ],
    compiler_params=pltpu.CompilerParams(
        needs_layout_passes=False, use_tc_tiling_on_sc=False),
)
def _scatter_arange(idx_hbm, out_hbm, idx_buf, local):
    wid = lax.axis_index("c") * NS + lax.axis_index("s")
    base = (wid * ROWS_PER_TILE).astype(jnp.int32)
    iota = lax.iota(jnp.int32, L)
    zeros = jnp.zeros((L,), jnp.int32)

    def _zero_body(j, _):
        local[pl.ds(j * L, L)] = zeros
        return 0

    lax.fori_loop(0, LOCAL_PAD // L, _zero_body, 0)

    limit = jnp.uint32(ROWS_PER_TILE)

    def _chunk_body(c, _):
        pltpu.sync_copy(idx_hbm.at[pl.ds(c * CHUNK, CHUNK)], idx_buf)
        chunk_base = (c * CHUNK).astype(jnp.int32)

        @plsc.parallel_loop(0, VPC, 1, unroll=8)
        def _vec_body(v):
            ivec = idx_buf[pl.ds(v * L, L)]
            val = (chunk_base + v * L) + iota
            loc = ivec - base
            m = plsc.bitcast(loc, jnp.uint32) < limit
            plsc.store_scatter(local, [loc], val, mask=m)

        return 0

    lax.fori_loop(0, NCHUNK, _chunk_body, 0)
    pltpu.sync_copy(local.at[pl.ds(0, ROWS_PER_TILE)], out_hbm.at[wid])


def kernel(X, row_missing_idx):
    del X
    return _scatter_arange(row_missing_idx).reshape(-1)


# speedup vs baseline: 24.5518x; 1.1071x over previous
import functools

import jax
import jax.numpy as jnp
from jax import lax
from jax.experimental import pallas as pl
from jax.experimental.pallas import tpu as pltpu
from jax.experimental.pallas import tpu_sc as plsc

N = 1_000_000
M = 500_000
NC = 2
NS = 16
NW = NC * NS
ROWS_PER_TILE = N // NW
LOCAL_PAD = 32_000
CHUNK = 10_000
NCHUNK = M // CHUNK
VPC = CHUNK // 16
L = 16

_mesh = plsc.VectorSubcoreMesh(core_axis_name="c", subcore_axis_name="s")


@functools.partial(
    pl.kernel,
    mesh=_mesh,
    out_type=jax.ShapeDtypeStruct((NW, ROWS_PER_TILE), jnp.int32),
    scratch_types=[
        pltpu.VMEM((CHUNK,), jnp.int32),
        pltpu.VMEM((CHUNK,), jnp.int32),
        pltpu.VMEM((LOCAL_PAD,), jnp.int32),
        pltpu.SemaphoreType.DMA,
        pltpu.SemaphoreType.DMA,
    ],
    compiler_params=pltpu.CompilerParams(
        needs_layout_passes=False, use_tc_tiling_on_sc=False),
)
def _scatter_arange(idx_hbm, out_hbm, buf_a, buf_b, local, sem_a, sem_b):
    wid = lax.axis_index("c") * NS + lax.axis_index("s")
    base = (wid * ROWS_PER_TILE).astype(jnp.int32)
    iota = lax.iota(jnp.int32, L)
    zeros = jnp.zeros((L,), jnp.int32)
    limit = jnp.uint32(ROWS_PER_TILE)

    pltpu.async_copy(idx_hbm.at[pl.ds(0, CHUNK)], buf_a, sem_a)

    @plsc.parallel_loop(0, LOCAL_PAD // L, 1, unroll=8)
    def _zero_body(j):
        local[pl.ds(j * L, L)] = zeros

    def _process(buf, chunk_base):
        @plsc.parallel_loop(0, VPC, 1, unroll=5)
        def _vec_body(v):
            ivec = buf[pl.ds(v * L, L)]
            val = (chunk_base + v * L) + iota
            loc = ivec - base
            m = plsc.bitcast(loc, jnp.uint32) < limit
            plsc.store_scatter(local, [loc], val, mask=m)

    def _pair_body(c, _):
        c0 = 2 * c
        pltpu.make_async_copy(idx_hbm.at[pl.ds(0, CHUNK)], buf_a, sem_a).wait()
        pltpu.async_copy(
            idx_hbm.at[pl.ds((c0 + 1) * CHUNK, CHUNK)], buf_b, sem_b)
        _process(buf_a, (c0 * CHUNK).astype(jnp.int32))
        nxt = jnp.minimum(c0 + 2, NCHUNK - 1)
        pltpu.make_async_copy(idx_hbm.at[pl.ds(0, CHUNK)], buf_b, sem_b).wait()
        pltpu.async_copy(idx_hbm.at[pl.ds(nxt * CHUNK, CHUNK)], buf_a, sem_a)
        _process(buf_b, ((c0 + 1) * CHUNK).astype(jnp.int32))
        return 0

    lax.fori_loop(0, NCHUNK // 2, _pair_body, 0)
    pltpu.make_async_copy(idx_hbm.at[pl.ds(0, CHUNK)], buf_a, sem_a).wait()
    pltpu.sync_copy(local.at[pl.ds(0, ROWS_PER_TILE)], out_hbm.at[wid])


def kernel(X, row_missing_idx):
    del X
    return _scatter_arange(row_missing_idx).reshape(-1)
